# Initial kernel scaffold; baseline (speedup 1.0000x reference)
import jax
import jax.numpy as jnp
from jax.experimental import pallas as pl


def kernel(x, edge_index, W_enc, b_enc, W1, b1, W2, b2, W3, b3, W_dec, b_dec):
    raise NotImplementedError("write your pallas kernel here")


# speedup vs baseline: 8.4328x; 8.4328x over previous
import functools
import jax
import jax.numpy as jnp
from jax import lax
from jax.experimental import pallas as pl
from jax.experimental.pallas import tpu as pltpu, tpu_sc as plsc

N = 10000
E = 320000
D = 128
NP = 10240
ROWS_PER_TILE = NP // 16
CH = 128
NCHUNK = E // CH
NW = 32
BASE_CNT = NCHUNK // NW
REM = NCHUNK - BASE_CNT * NW
ZR = 16
BR = 1024


def _mesh():
    return plsc.VectorSubcoreMesh(core_axis_name="c", subcore_axis_name="s",
                                  num_cores=2, num_subcores=16)


def _sc_body(gather, *refs):
    if gather:
        (t_hbm, src_hbm, dst_hbm, agg_out,
         shared_agg, zeros_v, idx_s, idx_d, rows_v, sem) = refs
    else:
        (dst_hbm, agg_out,
         shared_agg, zeros_v, idx_d, rows_v) = refs

    c = lax.axis_index("c")
    s = lax.axis_index("s")
    wid = s * 2 + c

    def fill(i, carry):
        for j in range(D // 16):
            zeros_v[i, pl.ds(j * 16, 16)] = jnp.zeros((16,), jnp.float32)
        return carry
    lax.fori_loop(0, ZR, fill, 0)

    if not gather:
        def fill_ones(i, carry):
            for j in range(D // 16):
                rows_v[i, pl.ds(j * 16, 16)] = jnp.ones((16,), jnp.float32)
            return carry
        lax.fori_loop(0, CH, fill_ones, 0)

    row0 = pl.multiple_of(s * ROWS_PER_TILE, ROWS_PER_TILE)
    for k in range(ROWS_PER_TILE // ZR):
        pltpu.sync_copy(zeros_v, shared_agg.at[pl.ds(row0 + k * ZR, ZR)])
    plsc.subcore_barrier()

    cnt = BASE_CNT + jnp.where(wid < REM, 1, 0)

    def ebody(k, carry):
        chunk = wid + NW * k
        base = pl.multiple_of(chunk * CH, CH)
        pltpu.sync_copy(dst_hbm.at[pl.ds(base, CH)], idx_d)
        if gather:
            pltpu.sync_copy(src_hbm.at[pl.ds(base, CH)], idx_s)
            pltpu.async_copy(t_hbm.at[idx_s], rows_v, sem).wait()
        pltpu.sync_copy(rows_v, shared_agg.at[idx_d], add=True)
        return carry
    lax.fori_loop(0, cnt, ebody, 0)

    plsc.subcore_barrier()
    pltpu.sync_copy(shared_agg.at[pl.ds(row0, ROWS_PER_TILE)],
                    agg_out.at[c, pl.ds(row0, ROWS_PER_TILE)])


@functools.lru_cache(maxsize=None)
def _make_sc(gather):
    if gather:
        scratch = [
            pltpu.VMEM_SHARED((NP, D), jnp.float32),
            pltpu.VMEM((ZR, D), jnp.float32),
            pltpu.VMEM((CH,), jnp.int32),
            pltpu.VMEM((CH,), jnp.int32),
            pltpu.VMEM((CH, D), jnp.float32),
            pltpu.SemaphoreType.DMA,
        ]
    else:
        scratch = [
            pltpu.VMEM_SHARED((NP, D), jnp.float32),
            pltpu.VMEM((ZR, D), jnp.float32),
            pltpu.VMEM((CH,), jnp.int32),
            pltpu.VMEM((CH, D), jnp.float32),
        ]
    return pl.kernel(
        functools.partial(_sc_body, gather),
        out_type=jax.ShapeDtypeStruct((2, NP, D), jnp.float32),
        mesh=_mesh(),
        scratch_types=scratch,
        name="sc_agg" if gather else "sc_mask")


def _enc_body(x_ref, we_ref, be_ref, o_ref):
    o_ref[...] = jnp.dot(x_ref[...], we_ref[...],
                         preferred_element_type=jnp.float32) + be_ref[...]


def _tc_encode(x, W_enc, b_enc):
    return pl.pallas_call(
        _enc_body,
        grid=(pl.cdiv(N, BR),),
        in_specs=[
            pl.BlockSpec((BR, D), lambda i: (i, 0)),
            pl.BlockSpec((D, D), lambda i: (0, 0)),
            pl.BlockSpec((1, D), lambda i: (0, 0)),
        ],
        out_specs=pl.BlockSpec((BR, D), lambda i: (i, 0)),
        out_shape=jax.ShapeDtypeStruct((N, D), jnp.float32),
    )(x, W_enc, b_enc)


def _agg_full(p_ref, m_ref, h_ref):
    ps = p_ref[0] + p_ref[1]
    mc = jnp.sum(m_ref[0] + m_ref[1], axis=1)
    return ps + jnp.where((mc > 0.0)[:, None], h_ref[...], 0.0)


def _mid_body(p_ref, m_ref, h_ref, w_ref, b_ref, o_ref):
    a = _agg_full(p_ref, m_ref, h_ref)
    o_ref[...] = jnp.maximum(
        jnp.dot(a, w_ref[...], preferred_element_type=jnp.float32)
        + b_ref[...], 0.0)


def _tc_mid(p, m, h, W, b):
    return pl.pallas_call(
        _mid_body,
        grid=(pl.cdiv(N, BR),),
        in_specs=[
            pl.BlockSpec((2, BR, D), lambda i: (0, i, 0)),
            pl.BlockSpec((2, BR, D), lambda i: (0, i, 0)),
            pl.BlockSpec((BR, D), lambda i: (i, 0)),
            pl.BlockSpec((D, D), lambda i: (0, 0)),
            pl.BlockSpec((1, D), lambda i: (0, 0)),
        ],
        out_specs=pl.BlockSpec((BR, D), lambda i: (i, 0)),
        out_shape=jax.ShapeDtypeStruct((N, D), jnp.float32),
    )(p, m, h, W, b)


def _fin_body(p_ref, m_ref, h_ref, w_ref, b_ref, wd_ref, bd_ref, o_ref):
    a = _agg_full(p_ref, m_ref, h_ref)
    h3 = jnp.maximum(
        jnp.dot(a, w_ref[...], preferred_element_type=jnp.float32)
        + b_ref[...], 0.0)
    o_ref[...] = jnp.dot(h3, wd_ref[...],
                         preferred_element_type=jnp.float32) + bd_ref[...]


def _tc_final(p, m, h, W3, b3, W_dec, b_dec):
    return pl.pallas_call(
        _fin_body,
        grid=(pl.cdiv(N, BR),),
        in_specs=[
            pl.BlockSpec((2, BR, D), lambda i: (0, i, 0)),
            pl.BlockSpec((2, BR, D), lambda i: (0, i, 0)),
            pl.BlockSpec((BR, D), lambda i: (i, 0)),
            pl.BlockSpec((D, D), lambda i: (0, 0)),
            pl.BlockSpec((1, D), lambda i: (0, 0)),
            pl.BlockSpec((D, 1), lambda i: (0, 0)),
            pl.BlockSpec((1, 1), lambda i: (0, 0)),
        ],
        out_specs=pl.BlockSpec((BR, 1), lambda i: (i, 0)),
        out_shape=jax.ShapeDtypeStruct((N, 1), jnp.float32),
    )(p, m, h, W3, b3, W_dec, b_dec)


@jax.jit
def _run(x, edge_index, W_enc, b_enc, W1, b1, W2, b2, W3, b3, W_dec, b_dec):
    src = edge_index[0]
    dst = edge_index[1]
    b_enc2 = b_enc.reshape(1, D)
    b1_2 = b1.reshape(1, D)
    b2_2 = b2.reshape(1, D)
    b3_2 = b3.reshape(1, D)
    bd_2 = b_dec.reshape(1, 1)

    m = _make_sc(False)(dst)
    x0 = _tc_encode(x, W_enc, b_enc2)
    p1 = _make_sc(True)(x0, src, dst)
    h1 = _tc_mid(p1, m, x0, W1, b1_2)
    p2 = _make_sc(True)(h1, src, dst)
    h2 = _tc_mid(p2, m, h1, W2, b2_2)
    p3 = _make_sc(True)(h2, src, dst)
    return _tc_final(p3, m, h2, W3, b3_2, W_dec, bd_2)


def kernel(x, edge_index, W_enc, b_enc, W1, b1, W2, b2, W3, b3, W_dec, b_dec):
    return _run(x, edge_index, W_enc, b_enc, W1, b1, W2, b2, W3, b3,
                W_dec, b_dec)
